# baseline (speedup 1.0000x reference)
import jax
import jax.numpy as jnp
from jax.experimental import pallas as pl
from jax.experimental.pallas import tpu as pltpu

E = 16
TOP_K = 2
H = 768
I = 256


def _moe_body(x_ref, xb_ref, wr_ref, wg_ref, wu_ref, wd_ref, out_ref,
              w1_ref, w2_ref, i1_ref, i2_ref):
    e = pl.program_id(0)

    @pl.when(e == 0)
    def _router():
        x = x_ref[...]
        logits = jax.lax.dot_general(
            x, wr_ref[...], (((1,), (1,)), ((), ())),
            preferred_element_type=jnp.float32)
        scores = jax.nn.sigmoid(logits)
        eidx = jax.lax.broadcasted_iota(jnp.int32, scores.shape, 1)
        m1 = jnp.max(scores, axis=1, keepdims=True)
        is1 = scores == m1
        i1 = jnp.min(jnp.where(is1, eidx, E), axis=1, keepdims=True)
        excl = eidx == i1
        s2 = jnp.where(excl, -jnp.inf, scores)
        m2 = jnp.max(s2, axis=1, keepdims=True)
        i2 = jnp.min(jnp.where(s2 == m2, eidx, E), axis=1, keepdims=True)
        denom = m1 + m2 + 1e-20
        w1_ref[...] = m1 / denom
        w2_ref[...] = m2 / denom
        i1_ref[...] = i1
        i2_ref[...] = i2

    xb = xb_ref[...]
    g = jax.lax.dot_general(xb, wg_ref[0], (((1,), (0,)), ((), ())),
                            preferred_element_type=jnp.float32)
    u = jax.lax.dot_general(xb, wu_ref[0], (((1,), (0,)), ((), ())),
                            preferred_element_type=jnp.float32)
    hmid = (g * jax.nn.sigmoid(g) * u).astype(jnp.bfloat16)
    d = jax.lax.dot_general(hmid, wd_ref[0], (((1,), (0,)), ((), ())),
                            preferred_element_type=jnp.float32)
    w_e = (jnp.where(i1_ref[...] == e, w1_ref[...], 0.0) +
           jnp.where(i2_ref[...] == e, w2_ref[...], 0.0))
    contrib = d * w_e

    @pl.when(e == 0)
    def _init():
        out_ref[...] = contrib

    @pl.when(e != 0)
    def _acc():
        out_ref[...] += contrib


def kernel(hidden_states, W_router, W_gate, W_up, W_down):
    b, s, h = hidden_states.shape
    T = b * s
    x = hidden_states.reshape(T, h)
    xb = x.astype(jnp.bfloat16)
    wg = W_gate.astype(jnp.bfloat16)
    wu = W_up.astype(jnp.bfloat16)
    wd = W_down.astype(jnp.bfloat16)
    out = pl.pallas_call(
        _moe_body,
        grid=(E,),
        in_specs=[
            pl.BlockSpec((T, H), lambda e: (0, 0)),
            pl.BlockSpec((T, H), lambda e: (0, 0)),
            pl.BlockSpec((E, H), lambda e: (0, 0)),
            pl.BlockSpec((1, H, I), lambda e: (e, 0, 0)),
            pl.BlockSpec((1, H, I), lambda e: (e, 0, 0)),
            pl.BlockSpec((1, I, H), lambda e: (e, 0, 0)),
        ],
        out_specs=pl.BlockSpec((T, H), lambda e: (0, 0)),
        out_shape=jax.ShapeDtypeStruct((T, H), jnp.float32),
        scratch_shapes=[
            pltpu.VMEM((T, 1), jnp.float32),
            pltpu.VMEM((T, 1), jnp.float32),
            pltpu.VMEM((T, 1), jnp.int32),
            pltpu.VMEM((T, 1), jnp.int32),
        ],
    )(x, xb, W_router, wg, wu, wd)
    return out.reshape(b, s, h)


# speedup vs baseline: 1.4322x; 1.4322x over previous
import jax
import jax.numpy as jnp
from jax.experimental import pallas as pl
from jax.experimental.pallas import tpu as pltpu

E = 16
TOP_K = 2
H = 768
I = 256


def _moe_body(x_ref, wr_ref, wg_ref, wu_ref, wd_ref, out_ref,
              w1_ref, w2_ref, i1_ref, i2_ref, xb_ref):
    e = pl.program_id(0)

    @pl.when(e == 0)
    def _cast_x():
        xb_ref[...] = x_ref[...].astype(jnp.bfloat16)

    @pl.when(e == 0)
    def _router():
        x = x_ref[...]
        logits = jax.lax.dot_general(
            x, wr_ref[...], (((1,), (1,)), ((), ())),
            preferred_element_type=jnp.float32)
        scores = jax.nn.sigmoid(logits)
        eidx = jax.lax.broadcasted_iota(jnp.int32, scores.shape, 1)
        m1 = jnp.max(scores, axis=1, keepdims=True)
        is1 = scores == m1
        i1 = jnp.min(jnp.where(is1, eidx, E), axis=1, keepdims=True)
        excl = eidx == i1
        s2 = jnp.where(excl, -jnp.inf, scores)
        m2 = jnp.max(s2, axis=1, keepdims=True)
        i2 = jnp.min(jnp.where(s2 == m2, eidx, E), axis=1, keepdims=True)
        denom = m1 + m2 + 1e-20
        w1_ref[...] = m1 / denom
        w2_ref[...] = m2 / denom
        i1_ref[...] = i1
        i2_ref[...] = i2

    xb = xb_ref[...]
    wg = wg_ref[0].astype(jnp.bfloat16)
    wu = wu_ref[0].astype(jnp.bfloat16)
    wd = wd_ref[0].astype(jnp.bfloat16)
    g = jax.lax.dot_general(xb, wg, (((1,), (0,)), ((), ())),
                            preferred_element_type=jnp.float32)
    u = jax.lax.dot_general(xb, wu, (((1,), (0,)), ((), ())),
                            preferred_element_type=jnp.float32)
    hmid = (g * jax.nn.sigmoid(g) * u).astype(jnp.bfloat16)
    d = jax.lax.dot_general(hmid, wd, (((1,), (0,)), ((), ())),
                            preferred_element_type=jnp.float32)
    w_e = (jnp.where(i1_ref[...] == e, w1_ref[...], 0.0) +
           jnp.where(i2_ref[...] == e, w2_ref[...], 0.0))
    contrib = d * w_e

    @pl.when(e == 0)
    def _init():
        out_ref[...] = contrib

    @pl.when(e != 0)
    def _acc():
        out_ref[...] += contrib


def kernel(hidden_states, W_router, W_gate, W_up, W_down):
    b, s, h = hidden_states.shape
    T = b * s
    x = hidden_states.reshape(T, h)
    out = pl.pallas_call(
        _moe_body,
        grid=(E,),
        in_specs=[
            pl.BlockSpec((T, H), lambda e: (0, 0)),
            pl.BlockSpec((E, H), lambda e: (0, 0)),
            pl.BlockSpec((1, H, I), lambda e: (e, 0, 0)),
            pl.BlockSpec((1, H, I), lambda e: (e, 0, 0)),
            pl.BlockSpec((1, I, H), lambda e: (e, 0, 0)),
        ],
        out_specs=pl.BlockSpec((T, H), lambda e: (0, 0)),
        out_shape=jax.ShapeDtypeStruct((T, H), jnp.float32),
        scratch_shapes=[
            pltpu.VMEM((T, 1), jnp.float32),
            pltpu.VMEM((T, 1), jnp.float32),
            pltpu.VMEM((T, 1), jnp.int32),
            pltpu.VMEM((T, 1), jnp.int32),
            pltpu.VMEM((T, H), jnp.bfloat16),
        ],
    )(x, W_router, W_gate, W_up, W_down)
    return out.reshape(b, s, h)


# speedup vs baseline: 1.6211x; 1.1319x over previous
import jax
import jax.numpy as jnp
from jax.experimental import pallas as pl
from jax.experimental.pallas import tpu as pltpu

E = 16
TOP_K = 2
H = 768
I = 256
MB = 4


def _moe_body(x_ref, wr_ref, wg_ref, wu_ref, wd_ref, out_ref,
              rw_ref, hmid_ref, wdb_ref):
    e = pl.program_id(0)
    T = x_ref.shape[0]

    @pl.when(e == 0)
    def _router():
        x = x_ref[...]
        logits = jax.lax.dot_general(
            x, wr_ref[...], (((1,), (1,)), ((), ())),
            preferred_element_type=jnp.float32)
        scores = jax.nn.sigmoid(logits)
        eidx = jax.lax.broadcasted_iota(jnp.int32, scores.shape, 1)
        m1 = jnp.max(scores, axis=1, keepdims=True)
        is1 = scores == m1
        i1 = jnp.min(jnp.where(is1, eidx, E), axis=1, keepdims=True)
        excl = eidx == i1
        s2 = jnp.where(excl, -jnp.inf, scores)
        m2 = jnp.max(s2, axis=1, keepdims=True)
        i2 = jnp.min(jnp.where(s2 == m2, eidx, E), axis=1, keepdims=True)
        denom = m1 + m2 + 1e-20
        w1 = m1 / denom
        w2 = m2 / denom
        rw_ref[...] = (jnp.where(eidx == i1, w1, 0.0) +
                       jnp.where(eidx == i2, w2, 0.0))

    @pl.when(e < E)
    def _expert():
        x = x_ref[...]
        g = jax.lax.dot_general(x, wg_ref[0], (((1,), (0,)), ((), ())),
                                preferred_element_type=jnp.float32)
        u = jax.lax.dot_general(x, wu_ref[0], (((1,), (0,)), ((), ())),
                                preferred_element_type=jnp.float32)
        eidx = jax.lax.broadcasted_iota(jnp.int32, (T, E), 1)
        w_e = jnp.sum(jnp.where(eidx == e, rw_ref[...], 0.0),
                      axis=1, keepdims=True)
        hmid = (g * jax.nn.sigmoid(g) * u * w_e).astype(jnp.bfloat16)
        wdb = wd_ref[0].astype(jnp.bfloat16)
        for k in range(E):
            @pl.when(e == k)
            def _store():
                hmid_ref[:, k * I:(k + 1) * I] = hmid
                wdb_ref[k * I:(k + 1) * I, :] = wdb

    @pl.when(e >= E)
    def _down():
        m = e - E
        rows = T // MB
        hm = hmid_ref[pl.ds(m * rows, rows), :]
        out_ref[...] = jax.lax.dot_general(
            hm, wdb_ref[...], (((1,), (0,)), ((), ())),
            preferred_element_type=jnp.float32)


def kernel(hidden_states, W_router, W_gate, W_up, W_down):
    b, s, h = hidden_states.shape
    T = b * s
    x = hidden_states.reshape(T, h)
    out = pl.pallas_call(
        _moe_body,
        grid=(E + MB,),
        in_specs=[
            pl.BlockSpec((T, H), lambda e: (0, 0)),
            pl.BlockSpec((E, H), lambda e: (0, 0)),
            pl.BlockSpec((1, H, I), lambda e: (jnp.minimum(e, E - 1), 0, 0)),
            pl.BlockSpec((1, H, I), lambda e: (jnp.minimum(e, E - 1), 0, 0)),
            pl.BlockSpec((1, I, H), lambda e: (jnp.minimum(e, E - 1), 0, 0)),
        ],
        out_specs=pl.BlockSpec(
            (T // MB, H), lambda e: (jnp.clip(e - E, 0, MB - 1), 0)),
        out_shape=jax.ShapeDtypeStruct((T, H), jnp.float32),
        scratch_shapes=[
            pltpu.VMEM((T, E), jnp.float32),
            pltpu.VMEM((T, E * I), jnp.bfloat16),
            pltpu.VMEM((E * I, H), jnp.bfloat16),
        ],
    )(x, W_router, W_gate, W_up, W_down)
    return out.reshape(b, s, h)


# speedup vs baseline: 1.6562x; 1.0217x over previous
import jax
import jax.numpy as jnp
from jax.experimental import pallas as pl
from jax.experimental.pallas import tpu as pltpu

E = 16
TOP_K = 2
H = 768
I = 256
EPB = 2
NE = E // EPB
MB = 4


def _moe_body(x_ref, wr_ref, wg_ref, wu_ref, wd_ref, out_ref,
              rw_ref, hmid_ref, wdb_ref):
    j = pl.program_id(0)
    T = x_ref.shape[0]

    @pl.when(j == 0)
    def _router():
        x = x_ref[...]
        logits = jax.lax.dot_general(
            x, wr_ref[...], (((1,), (1,)), ((), ())),
            preferred_element_type=jnp.float32)
        scores = jax.nn.sigmoid(logits)
        eidx = jax.lax.broadcasted_iota(jnp.int32, scores.shape, 1)
        m1 = jnp.max(scores, axis=1, keepdims=True)
        is1 = scores == m1
        i1 = jnp.min(jnp.where(is1, eidx, E), axis=1, keepdims=True)
        excl = eidx == i1
        s2 = jnp.where(excl, -jnp.inf, scores)
        m2 = jnp.max(s2, axis=1, keepdims=True)
        i2 = jnp.min(jnp.where(s2 == m2, eidx, E), axis=1, keepdims=True)
        denom = m1 + m2 + 1e-20
        w1 = m1 / denom
        w2 = m2 / denom
        rw_ref[...] = (jnp.where(eidx == i1, w1, 0.0) +
                       jnp.where(eidx == i2, w2, 0.0))

    @pl.when(j < NE)
    def _experts():
        x = x_ref[...]
        eidx = jax.lax.broadcasted_iota(jnp.int32, (T, E), 1)
        hmids = []
        for p in range(EPB):
            e = j * EPB + p
            g = jax.lax.dot_general(x, wg_ref[p], (((1,), (0,)), ((), ())),
                                    preferred_element_type=jnp.float32)
            u = jax.lax.dot_general(x, wu_ref[p], (((1,), (0,)), ((), ())),
                                    preferred_element_type=jnp.float32)
            w_e = jnp.sum(jnp.where(eidx == e, rw_ref[...], 0.0),
                          axis=1, keepdims=True)
            hmids.append((g * jax.nn.sigmoid(g) * u * w_e).astype(jnp.bfloat16))
        wdb = wd_ref[...].astype(jnp.bfloat16)
        for k in range(NE):
            @pl.when(j == k)
            def _store():
                base = k * EPB * I
                for p in range(EPB):
                    hmid_ref[:, base + p * I:base + (p + 1) * I] = hmids[p]
                    wdb_ref[base + p * I:base + (p + 1) * I, :] = wdb[p]

    @pl.when(j >= NE)
    def _down():
        m = j - NE
        rows = T // MB
        hm = hmid_ref[pl.ds(m * rows, rows), :]
        out_ref[...] = jax.lax.dot_general(
            hm, wdb_ref[...], (((1,), (0,)), ((), ())),
            preferred_element_type=jnp.float32)


def kernel(hidden_states, W_router, W_gate, W_up, W_down):
    b, s, h = hidden_states.shape
    T = b * s
    x = hidden_states.reshape(T, h)
    out = pl.pallas_call(
        _moe_body,
        grid=(NE + MB,),
        in_specs=[
            pl.BlockSpec((T, H), lambda j: (0, 0)),
            pl.BlockSpec((E, H), lambda j: (0, 0)),
            pl.BlockSpec((EPB, H, I),
                         lambda j: (jnp.minimum(j, NE - 1), 0, 0)),
            pl.BlockSpec((EPB, H, I),
                         lambda j: (jnp.minimum(j, NE - 1), 0, 0)),
            pl.BlockSpec((EPB, I, H),
                         lambda j: (jnp.minimum(j, NE - 1), 0, 0)),
        ],
        out_specs=pl.BlockSpec(
            (T // MB, H), lambda j: (jnp.clip(j - NE, 0, MB - 1), 0)),
        out_shape=jax.ShapeDtypeStruct((T, H), jnp.float32),
        scratch_shapes=[
            pltpu.VMEM((T, E), jnp.float32),
            pltpu.VMEM((T, E * I), jnp.bfloat16),
            pltpu.VMEM((E * I, H), jnp.bfloat16),
        ],
    )(x, W_router, W_gate, W_up, W_down)
    return out.reshape(b, s, h)
